# baseline (speedup 1.0000x reference)
import functools

import jax
import jax.numpy as jnp
from jax import lax
from jax.experimental import pallas as pl
from jax.experimental.pallas import tpu as pltpu
from jax.experimental.pallas import tpu_sc as plsc

_BLOCK_ROWS = 256
_TAU = 1.4
_NW = 32
_SLOTS = 16
_CAP = _NW * _SLOTS
_SENTINEL = 1 << 20


def _mm_kernel(w_ref, mt_ref, m_out_ref, b2_ref):
    mt = mt_ref[...]
    m_out_ref[...] = jnp.dot(w_ref[...], mt, preferred_element_type=jnp.float32)
    b2_ref[...] = jnp.sum(mt * mt, axis=0, keepdims=True)


def _scores_kernel(x_ref, m_ref, b2_ref, cidx_ref, gap_ref):
    scores = b2_ref[...] - 2.0 * jnp.dot(
        x_ref[...], m_ref[...], preferred_element_type=jnp.float32)
    c = scores.shape[1]
    col = jax.lax.broadcasted_iota(jnp.int32, scores.shape, 1)
    min1 = jnp.min(scores, axis=1, keepdims=True)
    idx = jnp.min(jnp.where(scores == min1, col, c), axis=1, keepdims=True)
    min2 = jnp.min(jnp.where(col == idx, jnp.inf, scores), axis=1,
                   keepdims=True)
    cidx_ref[...] = idx.reshape(1, -1, 1)
    gap_ref[...] = (min2 - min1).reshape(1, 1, -1)


def _exact_kernel(xs_ref, w_ref, mt_ref, eidx_ref):
    preds = jnp.dot(xs_ref[...], w_ref[...], preferred_element_type=jnp.float32)
    pm = jnp.dot(preds, mt_ref[...], preferred_element_type=jnp.float32)
    a2 = jnp.sum(preds * preds, axis=1, keepdims=True)
    b2 = jnp.sum(mt_ref[...] * mt_ref[...], axis=0, keepdims=True)
    d2 = a2 + b2 - 2.0 * pm
    dist = jnp.sqrt(jnp.clip(d2, 0.0, None))
    c = dist.shape[1]
    col = jax.lax.broadcasted_iota(jnp.int32, dist.shape, 1)
    min_d = jnp.min(dist, axis=1, keepdims=True)
    eidx_ref[...] = jnp.min(jnp.where(dist == min_d, col, c), axis=1,
                            keepdims=True)


def _onehot_kernel(fidx_ref, out_ref):
    idx = fidx_ref[0]
    col = jax.lax.broadcasted_iota(jnp.int32, out_ref.shape, 1)
    out_ref[...] = (col == idx).astype(jnp.float32)


def _sc_compact_gather_body(ns, d_in, rows_per_w,
                            gap_hbm, x_hbm, rowids_hbm, xsel_hbm,
                            gap_v, rid_v, gidx_v, rows_v, sem):
    c = lax.axis_index("c")
    s = lax.axis_index("s")
    wid = s * 2 + c
    base = wid * rows_per_w
    pltpu.sync_copy(gap_hbm.at[pl.ds(base, rows_per_w)], gap_v)
    rid_v[...] = jnp.full((16,), _SENTINEL, jnp.int32)
    cnt_v = jnp.zeros((16,), jnp.int32)
    for i in range(rows_per_w // 16):
        g = gap_v[pl.ds(i * 16, 16)]
        flags = g < _TAU
        fi = jnp.where(flags, jnp.int32(1), jnp.int32(0))
        incl = plsc.cumsum(fi)
        pos = jnp.minimum(cnt_v + incl - fi, _SLOTS - 1)
        rows = base + i * 16 + lax.iota(jnp.int32, 16)
        plsc.store_scatter(rid_v, [pos], rows, mask=flags)
        cnt_v = cnt_v + plsc.cummax(lax.rev(incl, (0,)))
    gidx_v[...] = jnp.minimum(rid_v[...], ns - 1)
    pltpu.async_copy(x_hbm.at[gidx_v], rows_v, sem).wait()
    pltpu.sync_copy(rid_v, rowids_hbm.at[pl.ds(wid * _SLOTS, _SLOTS)])
    pltpu.sync_copy(rows_v, xsel_hbm.at[pl.ds(wid * _SLOTS, _SLOTS)])


def _sc_compact_gather(gap, x):
    ns, d_in = x.shape
    rows_per_w = ns // _NW
    mesh = plsc.VectorSubcoreMesh(core_axis_name="c", subcore_axis_name="s")
    body = functools.partial(_sc_compact_gather_body, ns, d_in, rows_per_w)
    return pl.kernel(
        body,
        mesh=mesh,
        out_type=[jax.ShapeDtypeStruct((_CAP,), jnp.int32),
                  jax.ShapeDtypeStruct((_CAP, d_in), jnp.float32)],
        scratch_types=[pltpu.VMEM((rows_per_w,), jnp.float32),
                       pltpu.VMEM((16,), jnp.int32),
                       pltpu.VMEM((16,), jnp.int32),
                       pltpu.VMEM((_SLOTS, d_in), jnp.float32),
                       pltpu.SemaphoreType.DMA],
        compiler_params=pltpu.CompilerParams(needs_layout_passes=False),
    )(gap, x)


def kernel(x, W, mean_features):
    ns, d_in = x.shape
    nf = W.shape[1]
    c = mean_features.shape[0]
    mt = mean_features.T
    nblk = ns // _BLOCK_ROWS

    m_proj, b2 = pl.pallas_call(
        _mm_kernel,
        out_shape=(jax.ShapeDtypeStruct((nf, c), jnp.float32),
                   jax.ShapeDtypeStruct((1, c), jnp.float32)),
    )(W, mt)

    cidx, gap = pl.pallas_call(
        _scores_kernel,
        grid=(nblk,),
        in_specs=[
            pl.BlockSpec((_BLOCK_ROWS, nf), lambda i: (i, 0)),
            pl.BlockSpec((nf, c), lambda i: (0, 0)),
            pl.BlockSpec((1, c), lambda i: (0, 0)),
        ],
        out_specs=(pl.BlockSpec((1, _BLOCK_ROWS, 1), lambda i: (i, 0, 0)),
                   pl.BlockSpec((1, 1, _BLOCK_ROWS), lambda i: (i, 0, 0))),
        out_shape=(jax.ShapeDtypeStruct((nblk, _BLOCK_ROWS, 1), jnp.int32),
                   jax.ShapeDtypeStruct((nblk, 1, _BLOCK_ROWS), jnp.float32)),
        compiler_params=pltpu.CompilerParams(
            dimension_semantics=("parallel",)),
    )(x, m_proj, b2)

    rowids, x_sel = _sc_compact_gather(gap.reshape(ns), x)

    eidx = pl.pallas_call(
        _exact_kernel,
        out_shape=jax.ShapeDtypeStruct((_CAP, 1), jnp.int32),
    )(x_sel, W, mt)

    fidx = cidx.reshape(ns).at[rowids].set(eidx.reshape(_CAP), mode="drop")

    return pl.pallas_call(
        _onehot_kernel,
        grid=(nblk,),
        in_specs=[pl.BlockSpec((1, _BLOCK_ROWS, 1), lambda i: (i, 0, 0))],
        out_specs=pl.BlockSpec((_BLOCK_ROWS, c), lambda i: (i, 0)),
        out_shape=jax.ShapeDtypeStruct((ns, c), jnp.float32),
        compiler_params=pltpu.CompilerParams(
            dimension_semantics=("parallel",)),
    )(fidx.reshape(nblk, _BLOCK_ROWS, 1))


# speedup vs baseline: 1.0718x; 1.0718x over previous
import functools

import jax
import jax.numpy as jnp
from jax import lax
from jax.experimental import pallas as pl
from jax.experimental.pallas import tpu as pltpu
from jax.experimental.pallas import tpu_sc as plsc

_BLOCK_ROWS = 256
_TAU = 1.4
_NW = 32
_SLOTS = 16
_CAP = _NW * _SLOTS
_SENTINEL = 1 << 20


def _mm_kernel(w_ref, m_ref, m_out_ref, b2_ref):
    m = m_ref[...]
    dn = (((1,), (1,)), ((), ()))
    m_out_ref[...] = jax.lax.dot_general(
        w_ref[...], m, dn, preferred_element_type=jnp.float32)
    ones = jnp.ones((8, m.shape[1]), jnp.float32)
    b2_ref[...] = jax.lax.dot_general(
        ones, m * m, dn, preferred_element_type=jnp.float32)[:1]


def _scores_kernel(x_ref, m_ref, b2_ref, cidx_ref, gap_ref):
    scores = b2_ref[...] - 2.0 * jnp.dot(
        x_ref[...], m_ref[...], preferred_element_type=jnp.float32)
    c = scores.shape[1]
    col = jax.lax.broadcasted_iota(jnp.int32, scores.shape, 1)
    min1 = jnp.min(scores, axis=1, keepdims=True)
    idx = jnp.min(jnp.where(scores == min1, col, c), axis=1, keepdims=True)
    min2 = jnp.min(jnp.where(col == idx, jnp.inf, scores), axis=1,
                   keepdims=True)
    cidx_ref[...] = idx.reshape(1, -1, 1)
    gap_ref[...] = (min2 - min1).reshape(1, 1, -1)


def _exact_kernel(xs_ref, w_ref, m_ref, b2_ref, eidx_ref):
    preds = jnp.dot(xs_ref[...], w_ref[...], preferred_element_type=jnp.float32)
    pm = jax.lax.dot_general(preds, m_ref[...], (((1,), (1,)), ((), ())),
                             preferred_element_type=jnp.float32)
    a2 = jnp.sum(preds * preds, axis=1, keepdims=True)
    b2 = b2_ref[...]
    d2 = a2 + b2 - 2.0 * pm
    dist = jnp.sqrt(jnp.clip(d2, 0.0, None))
    c = dist.shape[1]
    col = jax.lax.broadcasted_iota(jnp.int32, dist.shape, 1)
    min_d = jnp.min(dist, axis=1, keepdims=True)
    eidx_ref[...] = jnp.min(jnp.where(dist == min_d, col, c), axis=1,
                            keepdims=True)


def _onehot_kernel(fidx_ref, out_ref):
    idx = fidx_ref[0]
    col = jax.lax.broadcasted_iota(jnp.int32, out_ref.shape, 1)
    out_ref[...] = (col == idx).astype(jnp.float32)


def _sc_compact_gather_body(ns, d_in, rows_per_w,
                            gap_hbm, x_hbm, rowids_hbm, xsel_hbm,
                            gap_v, rid_v, gidx_v, rows_v, sem):
    c = lax.axis_index("c")
    s = lax.axis_index("s")
    wid = s * 2 + c
    base = wid * rows_per_w
    pltpu.sync_copy(gap_hbm.at[pl.ds(base, rows_per_w)], gap_v)
    rid_v[...] = jnp.full((16,), _SENTINEL, jnp.int32)
    cnt_v = jnp.zeros((16,), jnp.int32)
    for i in range(rows_per_w // 16):
        g = gap_v[pl.ds(i * 16, 16)]
        flags = g < _TAU
        fi = jnp.where(flags, jnp.int32(1), jnp.int32(0))
        incl = plsc.cumsum(fi)
        pos = jnp.minimum(cnt_v + incl - fi, _SLOTS - 1)
        rows = base + i * 16 + lax.iota(jnp.int32, 16)
        plsc.store_scatter(rid_v, [pos], rows, mask=flags)
        cnt_v = cnt_v + plsc.cummax(lax.rev(incl, (0,)))
    gidx_v[...] = jnp.minimum(rid_v[...], ns - 1)
    pltpu.async_copy(x_hbm.at[gidx_v], rows_v, sem).wait()
    pltpu.sync_copy(rid_v, rowids_hbm.at[pl.ds(wid * _SLOTS, _SLOTS)])
    pltpu.sync_copy(rows_v, xsel_hbm.at[pl.ds(wid * _SLOTS, _SLOTS)])


def _sc_compact_gather(gap, x):
    ns, d_in = x.shape
    rows_per_w = ns // _NW
    mesh = plsc.VectorSubcoreMesh(core_axis_name="c", subcore_axis_name="s")
    body = functools.partial(_sc_compact_gather_body, ns, d_in, rows_per_w)
    return pl.kernel(
        body,
        mesh=mesh,
        out_type=[jax.ShapeDtypeStruct((_CAP,), jnp.int32),
                  jax.ShapeDtypeStruct((_CAP, d_in), jnp.float32)],
        scratch_types=[pltpu.VMEM((rows_per_w,), jnp.float32),
                       pltpu.VMEM((16,), jnp.int32),
                       pltpu.VMEM((16,), jnp.int32),
                       pltpu.VMEM((_SLOTS, d_in), jnp.float32),
                       pltpu.SemaphoreType.DMA],
        compiler_params=pltpu.CompilerParams(needs_layout_passes=False),
    )(gap, x)


def kernel(x, W, mean_features):
    ns, d_in = x.shape
    nf = W.shape[1]
    c = mean_features.shape[0]
    nblk = ns // _BLOCK_ROWS

    m_proj, b2 = pl.pallas_call(
        _mm_kernel,
        out_shape=(jax.ShapeDtypeStruct((nf, c), jnp.float32),
                   jax.ShapeDtypeStruct((1, c), jnp.float32)),
    )(W, mean_features)

    cidx, gap = pl.pallas_call(
        _scores_kernel,
        grid=(nblk,),
        in_specs=[
            pl.BlockSpec((_BLOCK_ROWS, nf), lambda i: (i, 0)),
            pl.BlockSpec((nf, c), lambda i: (0, 0)),
            pl.BlockSpec((1, c), lambda i: (0, 0)),
        ],
        out_specs=(pl.BlockSpec((1, _BLOCK_ROWS, 1), lambda i: (i, 0, 0)),
                   pl.BlockSpec((1, 1, _BLOCK_ROWS), lambda i: (i, 0, 0))),
        out_shape=(jax.ShapeDtypeStruct((nblk, _BLOCK_ROWS, 1), jnp.int32),
                   jax.ShapeDtypeStruct((nblk, 1, _BLOCK_ROWS), jnp.float32)),
        compiler_params=pltpu.CompilerParams(
            dimension_semantics=("parallel",)),
    )(x, m_proj, b2)

    rowids, x_sel = _sc_compact_gather(gap.reshape(ns), x)

    eidx = pl.pallas_call(
        _exact_kernel,
        out_shape=jax.ShapeDtypeStruct((_CAP, 1), jnp.int32),
    )(x_sel, W, mean_features, b2)

    fidx = cidx.reshape(ns).at[rowids].set(eidx.reshape(_CAP), mode="drop")

    return pl.pallas_call(
        _onehot_kernel,
        grid=(nblk,),
        in_specs=[pl.BlockSpec((1, _BLOCK_ROWS, 1), lambda i: (i, 0, 0))],
        out_specs=pl.BlockSpec((_BLOCK_ROWS, c), lambda i: (i, 0)),
        out_shape=jax.ShapeDtypeStruct((ns, c), jnp.float32),
        compiler_params=pltpu.CompilerParams(
            dimension_semantics=("parallel",)),
    )(fidx.reshape(nblk, _BLOCK_ROWS, 1))
